# Initial kernel scaffold; baseline (speedup 1.0000x reference)
import jax
import jax.numpy as jnp
from jax.experimental import pallas as pl


def kernel(user_emb, entity_emb, latent_emb, latent_div_emb, interact_mat, weight, weight_d, params_c, params_d, edge_index, edge_type, entity_cate_set):
    raise NotImplementedError("write your pallas kernel here")


# speedup vs baseline: 6.6470x; 6.6470x over previous
import functools

import jax
import jax.numpy as jnp
from jax import lax
from jax.experimental import pallas as pl
from jax.experimental.pallas import tpu as pltpu
from jax.experimental.pallas import tpu_sc as plsc

F32 = jnp.float32
I32 = jnp.int32

NENT = 10000
NUSR = 1024
NEDGE = 320000
EMB = 128
NF = 4
NREL = 10
NCAT = 20
R1 = NREL - 1
R2 = NCAT + NREL - 1
TEMP = 0.2

NC = 2
NS = 16
CHUNK = 80
NCH = NEDGE // CHUNK
IBATCH = 8
NBT = NCH // IBATCH
KMAX = (NBT + NS - 1) // NS
STRIPE = 640
LSTRIPE = NENT - (NS - 1) * STRIPE
BLK = 2000
UB = 128
NEB = NENT // BLK
NUB = NUSR // UB


def _sc_body(pc, pd, g0, g1, h2, ze, acc_c_o, acc_d_o,
             gidx, hidx, rows, acc_sh, sem):
    core = lax.axis_index("c")
    sid = lax.axis_index("s")
    row0 = sid * STRIPE
    last = NS - 1

    @pl.when(sid < last)
    def _():
        pltpu.sync_copy(ze, acc_sh.at[pl.ds(row0, STRIPE)])

    @pl.when(sid == last)
    def _():
        pltpu.sync_copy(ze.at[pl.ds(0, LSTRIPE)],
                        acc_sh.at[pl.ds(last * STRIPE, LSTRIPE)])

    plsc.subcore_barrier()

    def batch(k, carry):
        b = k * NS + sid

        @pl.when(b < NBT)
        def _():
            base = b * IBATCH

            @pl.when(core == 0)
            def _():
                pltpu.sync_copy(g0.at[pl.ds(base, IBATCH)], gidx)

            @pl.when(core == 1)
            def _():
                pltpu.sync_copy(g1.at[pl.ds(base, IBATCH)], gidx)

            pltpu.sync_copy(h2.at[pl.ds(base, IBATCH)], hidx)

            def chunk(j, c):
                @pl.when(core == 0)
                def _():
                    pltpu.async_copy(pc.at[gidx.at[j]], rows, sem).wait()

                @pl.when(core == 1)
                def _():
                    pltpu.async_copy(pd.at[gidx.at[j]], rows, sem).wait()

                pltpu.sync_copy(rows, acc_sh.at[hidx.at[j]], add=True)
                return c

            lax.fori_loop(0, IBATCH, chunk, 0)

        return carry

    lax.fori_loop(0, KMAX, batch, 0)
    plsc.subcore_barrier()

    @pl.when((core == 0) & (sid < last))
    def _():
        pltpu.sync_copy(acc_sh.at[pl.ds(row0, STRIPE)],
                        acc_c_o.at[pl.ds(row0, STRIPE)])

    @pl.when((core == 0) & (sid == last))
    def _():
        pltpu.sync_copy(acc_sh.at[pl.ds(last * STRIPE, LSTRIPE)],
                        acc_c_o.at[pl.ds(last * STRIPE, LSTRIPE)])

    @pl.when((core == 1) & (sid < last))
    def _():
        pltpu.sync_copy(acc_sh.at[pl.ds(row0, STRIPE)],
                        acc_d_o.at[pl.ds(row0, STRIPE)])

    @pl.when((core == 1) & (sid == last))
    def _():
        pltpu.sync_copy(acc_sh.at[pl.ds(last * STRIPE, LSTRIPE)],
                        acc_d_o.at[pl.ds(last * STRIPE, LSTRIPE)])


NBTC = NBT // NC
KMAXC = (NBTC + NS - 1) // NS


def _sc_cnt_body(h2, zc, on, cnt0_o, cnt1_o, hidx, ones, cnt_sh):
    core = lax.axis_index("c")
    sid = lax.axis_index("s")
    row0 = sid * STRIPE
    last = NS - 1

    @pl.when(sid < last)
    def _():
        pltpu.sync_copy(zc, cnt_sh.at[pl.ds(row0, STRIPE)])

    @pl.when(sid == last)
    def _():
        pltpu.sync_copy(zc.at[pl.ds(0, LSTRIPE)],
                        cnt_sh.at[pl.ds(last * STRIPE, LSTRIPE)])

    pltpu.sync_copy(on, ones)
    plsc.subcore_barrier()

    def batch(k, carry):
        kk = k * NS + sid

        @pl.when(kk < NBTC)
        def _():
            b = core * NBTC + kk
            pltpu.sync_copy(h2.at[pl.ds(b * IBATCH, IBATCH)], hidx)

            def chunk(j, c):
                pltpu.sync_copy(ones, cnt_sh.at[hidx.at[j]], add=True)
                return c

            lax.fori_loop(0, IBATCH, chunk, 0)

        return carry

    lax.fori_loop(0, KMAXC, batch, 0)
    plsc.subcore_barrier()

    @pl.when((core == 0) & (sid < last))
    def _():
        pltpu.sync_copy(cnt_sh.at[pl.ds(row0, STRIPE)],
                        cnt0_o.at[pl.ds(row0, STRIPE)])

    @pl.when((core == 0) & (sid == last))
    def _():
        pltpu.sync_copy(cnt_sh.at[pl.ds(last * STRIPE, LSTRIPE)],
                        cnt0_o.at[pl.ds(last * STRIPE, LSTRIPE)])

    @pl.when((core == 1) & (sid < last))
    def _():
        pltpu.sync_copy(cnt_sh.at[pl.ds(row0, STRIPE)],
                        cnt1_o.at[pl.ds(row0, STRIPE)])

    @pl.when((core == 1) & (sid == last))
    def _():
        pltpu.sync_copy(cnt_sh.at[pl.ds(last * STRIPE, LSTRIPE)],
                        cnt1_o.at[pl.ds(last * STRIPE, LSTRIPE)])


def _make_sc():
    mesh = plsc.VectorSubcoreMesh(core_axis_name="c", subcore_axis_name="s",
                                  num_cores=NC, num_subcores=NS)
    outs = (jax.ShapeDtypeStruct((NENT, EMB), F32),
            jax.ShapeDtypeStruct((NENT, EMB), F32))
    scratch = [pltpu.VMEM((IBATCH, CHUNK), I32),
               pltpu.VMEM((IBATCH, CHUNK), I32),
               pltpu.VMEM((CHUNK, EMB), F32),
               pltpu.VMEM_SHARED((NENT, EMB), F32),
               pltpu.SemaphoreType.DMA]
    return pl.kernel(_sc_body, out_type=outs, mesh=mesh,
                     scratch_types=scratch)


def _make_sc_cnt():
    mesh = plsc.VectorSubcoreMesh(core_axis_name="c", subcore_axis_name="s",
                                  num_cores=NC, num_subcores=NS)
    outs = (jax.ShapeDtypeStruct((NENT, EMB), F32),
            jax.ShapeDtypeStruct((NENT, EMB), F32))
    scratch = [pltpu.VMEM((IBATCH, CHUNK), I32),
               pltpu.VMEM((CHUNK, EMB), F32),
               pltpu.VMEM_SHARED((NENT, EMB), F32)]
    return pl.kernel(_sc_cnt_body, out_type=outs, mesh=mesh,
                     scratch_types=scratch)


def _mm(a, b):
    return lax.dot_general(a, b, (((1,), (0,)), ((), ())),
                           preferred_element_type=F32)


def _mmT(a, b):
    return lax.dot_general(a, b, (((1,), (1,)), ((), ())),
                           preferred_element_type=F32)


def _l2(x):
    return x / jnp.clip(jnp.sqrt(jnp.sum(x * x, axis=1, keepdims=True)),
                        1e-12, None)


def _leaky(x):
    return jnp.where(x >= 0, x, 0.2 * x)


def _softmax(x):
    m = jnp.max(x, axis=-1, keepdims=True)
    e = jnp.exp(x - m)
    return e / jnp.sum(e, axis=-1, keepdims=True)


def _onehot_wd(cate, wd):
    oh = (cate - 1 == lax.broadcasted_iota(I32, (cate.shape[0], R2), 1))
    return _mm(oh.astype(F32), wd)


def _prep_body(e_ref, w_ref, wd_ref, cate_ref, pc_ref, pd_ref):
    e = e_ref[...]
    pc_ref[...] = e * w_ref[0]
    pd_ref[...] = e * _onehot_wd(cate_ref[...], wd_ref[...])


_tc_prep = pl.pallas_call(
    _prep_body,
    grid=(R1, NEB),
    in_specs=[pl.BlockSpec((BLK, EMB), lambda r, i: (i, 0)),
              pl.BlockSpec((1, 1, EMB), lambda r, i: (r, 0, 0)),
              pl.BlockSpec((R2, EMB), lambda r, i: (0, 0)),
              pl.BlockSpec((BLK, 1), lambda r, i: (i, 0))],
    out_specs=[pl.BlockSpec((BLK, EMB), lambda r, i: (r * NEB + i, 0)),
               pl.BlockSpec((BLK, EMB), lambda r, i: (i, 0))],
    out_shape=[jax.ShapeDtypeStruct((R1 * NENT, EMB), F32),
               jax.ShapeDtypeStruct((NENT, EMB), F32)],
)


def _ent0_body(acc_c_ref, acc_d_ref, cnt0_ref, cnt1_ref, w_ref, wd_ref,
               cate_ref, ec_ref, ed_ref, pc_ref, pd_ref):
    c = jnp.clip(cnt0_ref[...][:, :1] + cnt1_ref[...][:, :1], 1.0, None)
    ec = _l2(acc_c_ref[...] / c)
    ed = _l2(acc_d_ref[...] / c)
    ec_ref[...] = ec
    ed_ref[...] = ed
    pc_ref[...] = ec * w_ref[0]
    pd_ref[...] = ed * _onehot_wd(cate_ref[...], wd_ref[...])


_tc_ent0 = pl.pallas_call(
    _ent0_body,
    grid=(R1, NEB),
    in_specs=[pl.BlockSpec((BLK, EMB), lambda r, i: (i, 0)),
              pl.BlockSpec((BLK, EMB), lambda r, i: (i, 0)),
              pl.BlockSpec((BLK, EMB), lambda r, i: (i, 0)),
              pl.BlockSpec((BLK, EMB), lambda r, i: (i, 0)),
              pl.BlockSpec((1, 1, EMB), lambda r, i: (r, 0, 0)),
              pl.BlockSpec((R2, EMB), lambda r, i: (0, 0)),
              pl.BlockSpec((BLK, 1), lambda r, i: (i, 0))],
    out_specs=[pl.BlockSpec((BLK, EMB), lambda r, i: (i, 0)),
               pl.BlockSpec((BLK, EMB), lambda r, i: (i, 0)),
               pl.BlockSpec((BLK, EMB), lambda r, i: (r * NEB + i, 0)),
               pl.BlockSpec((BLK, EMB), lambda r, i: (i, 0))],
    out_shape=[jax.ShapeDtypeStruct((NENT, EMB), F32),
               jax.ShapeDtypeStruct((NENT, EMB), F32),
               jax.ShapeDtypeStruct((R1 * NENT, EMB), F32),
               jax.ShapeDtypeStruct((NENT, EMB), F32)],
)


def _ent1_body(acc_c_ref, acc_d_ref, cnt0_ref, cnt1_ref, ent_ref, ec1_ref,
               ed1_ref, out_ref):
    c = jnp.clip(cnt0_ref[...][:, :1] + cnt1_ref[...][:, :1], 1.0, None)
    out_ref[...] = (2.0 * ent_ref[...] + ec1_ref[...] + ed1_ref[...]
                    + _l2(acc_c_ref[...] / c) + _l2(acc_d_ref[...] / c))


_tc_ent1 = pl.pallas_call(
    _ent1_body,
    grid=(NEB,),
    in_specs=[pl.BlockSpec((BLK, EMB), lambda i: (i, 0))] * 7,
    out_specs=pl.BlockSpec((BLK, EMB), lambda i: (i, 0)),
    out_shape=jax.ShapeDtypeStruct((NENT, EMB), F32),
)


def _user_branch(ua, u_prev, lat, w, p):
    w1, b1, uaW, uab, w2, b2, waW, wab = p
    t1u = _mmT(u_prev, w1) + b1
    t1l = _mmT(lat, w1) + b1
    score = _softmax(_leaky(_mmT(_mmT(t1u, t1l), uaW) + uab))
    t2l = _mmT(lat, w2) + b2
    t2w = _mmT(w, w2) + b2
    sa = _mmT(_mmT(t2l, t2w), waW) + wab
    lat_new = _mm(_softmax(_leaky(sa)), w)
    gate = _mm(score, lat_new)
    return _l2(ua * gate + ua), lat_new


def _user0_body(im_ref, e_ref, u_ref, latc_ref, latd_ref, w_ref, wd_ref,
                *rest):
    pc = [r[...] for r in rest[:8]]
    pd = [r[...] for r in rest[8:16]]
    uc_ref, ud_ref, lc_ref, ld_ref, cor_ref = rest[16:]
    ua = _mm(im_ref[...], e_ref[...])
    u_prev = u_ref[...]
    uc, lc = _user_branch(ua, u_prev, latc_ref[...], w_ref[...], pc)
    ud, ld = _user_branch(ua, u_prev, latd_ref[...], wd_ref[...], pd)
    uc_ref[...] = uc
    ud_ref[...] = ud
    lc_ref[...] = lc
    ld_ref[...] = ld
    w = w_ref[...]
    wT = w.T
    nrm = jnp.sqrt(jnp.sum(wT * wT, axis=1, keepdims=True))
    ndT = wT / nrm
    pos = jnp.sum(ndT * ndT, axis=1)
    ttl = jnp.sum(_mm(wT, w), axis=1)
    cor_ref[...] = jnp.full((1, 1), 0.0, F32) + jnp.sum(ttl - pos) / TEMP


def _pspecs():
    return [pl.BlockSpec()] * 8


_tc_user0 = pl.pallas_call(
    _user0_body,
    grid=(NUB,),
    in_specs=[pl.BlockSpec((UB, NENT), lambda i: (i, 0)),
              pl.BlockSpec((NENT, EMB), lambda i: (0, 0)),
              pl.BlockSpec((UB, EMB), lambda i: (i, 0)),
              pl.BlockSpec((NF, EMB), lambda i: (0, 0)),
              pl.BlockSpec((NF, EMB), lambda i: (0, 0)),
              pl.BlockSpec((R1, EMB), lambda i: (0, 0)),
              pl.BlockSpec((R2, EMB), lambda i: (0, 0))]
    + _pspecs() + _pspecs(),
    out_specs=[pl.BlockSpec((UB, EMB), lambda i: (i, 0)),
               pl.BlockSpec((UB, EMB), lambda i: (i, 0)),
               pl.BlockSpec((NF, EMB), lambda i: (0, 0)),
               pl.BlockSpec((NF, EMB), lambda i: (0, 0)),
               pl.BlockSpec((1, 1), lambda i: (0, 0))],
    out_shape=[jax.ShapeDtypeStruct((NUSR, EMB), F32),
               jax.ShapeDtypeStruct((NUSR, EMB), F32),
               jax.ShapeDtypeStruct((NF, EMB), F32),
               jax.ShapeDtypeStruct((NF, EMB), F32),
               jax.ShapeDtypeStruct((1, 1), F32)],
)


def _user1_body(im_ref, ec_ref, ed_ref, uc1_ref, ud1_ref, lc1_ref, ld1_ref,
                w_ref, wd_ref, uemb_ref, *rest):
    pc = [r[...] for r in rest[:8]]
    pd = [r[...] for r in rest[8:16]]
    out_ref = rest[16]
    im = im_ref[...]
    uac = _mm(im, ec_ref[...])
    uad = _mm(im, ed_ref[...])
    uc2, _ = _user_branch(uac, uc1_ref[...], lc1_ref[...], w_ref[...], pc)
    ud2, _ = _user_branch(uad, ud1_ref[...], ld1_ref[...], wd_ref[...], pd)
    out_ref[...] = (2.0 * uemb_ref[...] + uc1_ref[...] + ud1_ref[...]
                    + uc2 + ud2)


_tc_user1 = pl.pallas_call(
    _user1_body,
    grid=(NUB,),
    in_specs=[pl.BlockSpec((UB, NENT), lambda i: (i, 0)),
              pl.BlockSpec((NENT, EMB), lambda i: (0, 0)),
              pl.BlockSpec((NENT, EMB), lambda i: (0, 0)),
              pl.BlockSpec((UB, EMB), lambda i: (i, 0)),
              pl.BlockSpec((UB, EMB), lambda i: (i, 0)),
              pl.BlockSpec((NF, EMB), lambda i: (0, 0)),
              pl.BlockSpec((NF, EMB), lambda i: (0, 0)),
              pl.BlockSpec((R1, EMB), lambda i: (0, 0)),
              pl.BlockSpec((R2, EMB), lambda i: (0, 0)),
              pl.BlockSpec((UB, EMB), lambda i: (i, 0))]
    + _pspecs() + _pspecs(),
    out_specs=pl.BlockSpec((UB, EMB), lambda i: (i, 0)),
    out_shape=jax.ShapeDtypeStruct((NUSR, EMB), F32),
)


def _params(p, i):
    return (p["w1_W"][i], p["w1_b"][i].reshape(1, EMB),
            p["ua_W"][i], p["ua_b"][i].reshape(1, NF),
            p["w2_W"][i], p["w2_b"][i].reshape(1, EMB),
            p["wa_W"][i], p["wa_b"][i].reshape(1, -1))


def kernel(user_emb, entity_emb, latent_emb, latent_div_emb, interact_mat,
           weight, weight_d, params_c, params_d, edge_index, edge_type,
           entity_cate_set):
    head = edge_index[0]
    tail = edge_index[1]
    g0 = ((edge_type.astype(I32) - 1) * NENT + tail).reshape(NCH, CHUNK)
    g1 = tail.reshape(NCH, CHUNK)
    h2 = head.reshape(NCH, CHUNK)
    cate2 = entity_cate_set.reshape(NENT, 1)
    ze = jnp.zeros((STRIPE, EMB), F32)
    on = jnp.ones((CHUNK, EMB), F32)

    sc = _make_sc()
    sc_cnt = _make_sc_cnt()

    w3 = weight.reshape(R1, 1, EMB)
    pc0, pd0 = _tc_prep(entity_emb, w3, weight_d, cate2)
    cnt0, cnt1 = sc_cnt(h2, ze, on)
    acc_c0, acc_d0 = sc(pc0, pd0, g0, g1, h2, ze)
    e_c1, e_d1, pc1, pd1 = _tc_ent0(acc_c0, acc_d0, cnt0, cnt1, w3, weight_d,
                                    cate2)
    u_c1, u_d1, l_c1, l_d1, cor = _tc_user0(
        interact_mat, entity_emb, user_emb, latent_emb, latent_div_emb,
        weight, weight_d, *_params(params_c, 0), *_params(params_d, 0))
    acc_c1, acc_d1 = sc(pc1, pd1, g0, g1, h2, ze)
    e_fin = _tc_ent1(acc_c1, acc_d1, cnt0, cnt1, entity_emb, e_c1, e_d1)
    u_fin = _tc_user1(interact_mat, e_c1, e_d1, u_c1, u_d1, l_c1, l_d1,
                      weight, weight_d, user_emb,
                      *_params(params_c, 1), *_params(params_d, 1))
    return (e_fin, u_fin, cor[0, 0])


# speedup vs baseline: 8.8802x; 1.3360x over previous
import functools

import jax
import jax.numpy as jnp
from jax import lax
from jax.experimental import pallas as pl
from jax.experimental.pallas import tpu as pltpu
from jax.experimental.pallas import tpu_sc as plsc

F32 = jnp.float32
I32 = jnp.int32

NENT = 10000
NUSR = 1024
NEDGE = 320000
EMB = 128
NF = 4
NREL = 10
NCAT = 20
R1 = NREL - 1
R2 = NCAT + NREL - 1
TEMP = 0.2

NC = 2
NS = 16
CHUNK = 80
NCH = NEDGE // CHUNK
IBATCH = 8
NBT = NCH // IBATCH
KMAX = (NBT + NS - 1) // NS
STRIPE = 640
LSTRIPE = NENT - (NS - 1) * STRIPE
BLK = 2000
UB = 128
NEB = NENT // BLK
NUB = NUSR // UB


def _sc_body(tbl, g0, g1, h2, ze, acc_c_o, acc_d_o,
             gidx, hidx, rows0, rows1, acc_sh, sem0, sem1):
    core = lax.axis_index("c")
    sid = lax.axis_index("s")
    row0 = sid * STRIPE
    last = NS - 1

    @pl.when(sid < last)
    def _():
        pltpu.sync_copy(ze, acc_sh.at[pl.ds(row0, STRIPE)])

    @pl.when(sid == last)
    def _():
        pltpu.sync_copy(ze.at[pl.ds(0, LSTRIPE)],
                        acc_sh.at[pl.ds(last * STRIPE, LSTRIPE)])

    plsc.subcore_barrier()
    rows = (rows0, rows1)
    sems = (sem0, sem1)

    def batch(k, carry):
        b = k * NS + sid

        @pl.when(b < NBT)
        def _():
            base = b * IBATCH

            @pl.when(core == 0)
            def _():
                pltpu.sync_copy(g0.at[pl.ds(base, IBATCH)], gidx)

            @pl.when(core == 1)
            def _():
                pltpu.sync_copy(g1.at[pl.ds(base, IBATCH)], gidx)

            pltpu.sync_copy(h2.at[pl.ds(base, IBATCH)], hidx)

            descs = [None, None]
            descs[0] = pltpu.async_copy(tbl.at[gidx.at[0]], rows0, sem0)
            for j in range(IBATCH):
                if j + 1 < IBATCH:
                    descs[(j + 1) % 2] = pltpu.async_copy(
                        tbl.at[gidx.at[j + 1]], rows[(j + 1) % 2],
                        sems[(j + 1) % 2])
                descs[j % 2].wait()
                pltpu.sync_copy(rows[j % 2], acc_sh.at[hidx.at[j]], add=True)

        return carry

    lax.fori_loop(0, KMAX, batch, 0)
    plsc.subcore_barrier()

    @pl.when((core == 0) & (sid < last))
    def _():
        pltpu.sync_copy(acc_sh.at[pl.ds(row0, STRIPE)],
                        acc_c_o.at[pl.ds(row0, STRIPE)])

    @pl.when((core == 0) & (sid == last))
    def _():
        pltpu.sync_copy(acc_sh.at[pl.ds(last * STRIPE, LSTRIPE)],
                        acc_c_o.at[pl.ds(last * STRIPE, LSTRIPE)])

    @pl.when((core == 1) & (sid < last))
    def _():
        pltpu.sync_copy(acc_sh.at[pl.ds(row0, STRIPE)],
                        acc_d_o.at[pl.ds(row0, STRIPE)])

    @pl.when((core == 1) & (sid == last))
    def _():
        pltpu.sync_copy(acc_sh.at[pl.ds(last * STRIPE, LSTRIPE)],
                        acc_d_o.at[pl.ds(last * STRIPE, LSTRIPE)])


NBTC = NBT // NC
KMAXC = (NBTC + NS - 1) // NS


def _sc_cnt_body(h2, zc, on, cnt0_o, cnt1_o, hidx, ones, cnt_sh):
    core = lax.axis_index("c")
    sid = lax.axis_index("s")
    row0 = sid * STRIPE
    last = NS - 1

    @pl.when(sid < last)
    def _():
        pltpu.sync_copy(zc, cnt_sh.at[pl.ds(row0, STRIPE)])

    @pl.when(sid == last)
    def _():
        pltpu.sync_copy(zc.at[pl.ds(0, LSTRIPE)],
                        cnt_sh.at[pl.ds(last * STRIPE, LSTRIPE)])

    pltpu.sync_copy(on, ones)
    plsc.subcore_barrier()

    def batch(k, carry):
        kk = k * NS + sid

        @pl.when(kk < NBTC)
        def _():
            b = core * NBTC + kk
            pltpu.sync_copy(h2.at[pl.ds(b * IBATCH, IBATCH)], hidx)

            def chunk(j, c):
                pltpu.sync_copy(ones, cnt_sh.at[hidx.at[j]], add=True)
                return c

            lax.fori_loop(0, IBATCH, chunk, 0)

        return carry

    lax.fori_loop(0, KMAXC, batch, 0)
    plsc.subcore_barrier()

    @pl.when((core == 0) & (sid < last))
    def _():
        pltpu.sync_copy(cnt_sh.at[pl.ds(row0, STRIPE)],
                        cnt0_o.at[pl.ds(row0, STRIPE)])

    @pl.when((core == 0) & (sid == last))
    def _():
        pltpu.sync_copy(cnt_sh.at[pl.ds(last * STRIPE, LSTRIPE)],
                        cnt0_o.at[pl.ds(last * STRIPE, LSTRIPE)])

    @pl.when((core == 1) & (sid < last))
    def _():
        pltpu.sync_copy(cnt_sh.at[pl.ds(row0, STRIPE)],
                        cnt1_o.at[pl.ds(row0, STRIPE)])

    @pl.when((core == 1) & (sid == last))
    def _():
        pltpu.sync_copy(cnt_sh.at[pl.ds(last * STRIPE, LSTRIPE)],
                        cnt1_o.at[pl.ds(last * STRIPE, LSTRIPE)])


def _make_sc():
    mesh = plsc.VectorSubcoreMesh(core_axis_name="c", subcore_axis_name="s",
                                  num_cores=NC, num_subcores=NS)
    outs = (jax.ShapeDtypeStruct((NENT, EMB), F32),
            jax.ShapeDtypeStruct((NENT, EMB), F32))
    scratch = [pltpu.VMEM((IBATCH, CHUNK), I32),
               pltpu.VMEM((IBATCH, CHUNK), I32),
               pltpu.VMEM((CHUNK, EMB), F32),
               pltpu.VMEM((CHUNK, EMB), F32),
               pltpu.VMEM_SHARED((NENT, EMB), F32),
               pltpu.SemaphoreType.DMA,
               pltpu.SemaphoreType.DMA]
    return pl.kernel(_sc_body, out_type=outs, mesh=mesh,
                     scratch_types=scratch)


def _make_sc_cnt():
    mesh = plsc.VectorSubcoreMesh(core_axis_name="c", subcore_axis_name="s",
                                  num_cores=NC, num_subcores=NS)
    outs = (jax.ShapeDtypeStruct((NENT, EMB), F32),
            jax.ShapeDtypeStruct((NENT, EMB), F32))
    scratch = [pltpu.VMEM((IBATCH, CHUNK), I32),
               pltpu.VMEM((CHUNK, EMB), F32),
               pltpu.VMEM_SHARED((NENT, EMB), F32)]
    return pl.kernel(_sc_cnt_body, out_type=outs, mesh=mesh,
                     scratch_types=scratch)


def _mm(a, b):
    return lax.dot_general(a, b, (((1,), (0,)), ((), ())),
                           preferred_element_type=F32)


def _mmT(a, b):
    return lax.dot_general(a, b, (((1,), (1,)), ((), ())),
                           preferred_element_type=F32)


def _l2(x):
    return x / jnp.clip(jnp.sqrt(jnp.sum(x * x, axis=1, keepdims=True)),
                        1e-12, None)


def _leaky(x):
    return jnp.where(x >= 0, x, 0.2 * x)


def _softmax(x):
    m = jnp.max(x, axis=-1, keepdims=True)
    e = jnp.exp(x - m)
    return e / jnp.sum(e, axis=-1, keepdims=True)


def _onehot_wd(cate, wd):
    oh = (cate - 1 == lax.broadcasted_iota(I32, (cate.shape[0], R2), 1))
    return _mm(oh.astype(F32), wd)


NPT = (R1 + 1) * NENT


def _prep_body(e_ref, w_ref, wd_ref, cate_ref, p_ref):
    r = pl.program_id(0)
    e = e_ref[...]

    @pl.when(r < R1)
    def _():
        p_ref[...] = e * w_ref[0]

    @pl.when(r == R1)
    def _():
        p_ref[...] = e * _onehot_wd(cate_ref[...], wd_ref[...])


_tc_prep = pl.pallas_call(
    _prep_body,
    grid=(R1 + 1, NEB),
    in_specs=[pl.BlockSpec((BLK, EMB), lambda r, i: (i, 0)),
              pl.BlockSpec((1, 1, EMB), lambda r, i: (jnp.minimum(r, R1 - 1),
                                                      0, 0)),
              pl.BlockSpec((R2, EMB), lambda r, i: (0, 0)),
              pl.BlockSpec((BLK, 1), lambda r, i: (i, 0))],
    out_specs=pl.BlockSpec((BLK, EMB), lambda r, i: (r * NEB + i, 0)),
    out_shape=jax.ShapeDtypeStruct((NPT, EMB), F32),
)


def _ent0_body(acc_c_ref, acc_d_ref, cnt0_ref, cnt1_ref, w_ref, wd_ref,
               cate_ref, ec_ref, ed_ref, p_ref):
    r = pl.program_id(0)
    c = jnp.clip(cnt0_ref[...][:, :1] + cnt1_ref[...][:, :1], 1.0, None)
    ec = _l2(acc_c_ref[...] / c)
    ed = _l2(acc_d_ref[...] / c)
    ec_ref[...] = ec
    ed_ref[...] = ed

    @pl.when(r < R1)
    def _():
        p_ref[...] = ec * w_ref[0]

    @pl.when(r == R1)
    def _():
        p_ref[...] = ed * _onehot_wd(cate_ref[...], wd_ref[...])


_tc_ent0 = pl.pallas_call(
    _ent0_body,
    grid=(R1 + 1, NEB),
    in_specs=[pl.BlockSpec((BLK, EMB), lambda r, i: (i, 0)),
              pl.BlockSpec((BLK, EMB), lambda r, i: (i, 0)),
              pl.BlockSpec((BLK, EMB), lambda r, i: (i, 0)),
              pl.BlockSpec((BLK, EMB), lambda r, i: (i, 0)),
              pl.BlockSpec((1, 1, EMB), lambda r, i: (jnp.minimum(r, R1 - 1),
                                                      0, 0)),
              pl.BlockSpec((R2, EMB), lambda r, i: (0, 0)),
              pl.BlockSpec((BLK, 1), lambda r, i: (i, 0))],
    out_specs=[pl.BlockSpec((BLK, EMB), lambda r, i: (i, 0)),
               pl.BlockSpec((BLK, EMB), lambda r, i: (i, 0)),
               pl.BlockSpec((BLK, EMB), lambda r, i: (r * NEB + i, 0))],
    out_shape=[jax.ShapeDtypeStruct((NENT, EMB), F32),
               jax.ShapeDtypeStruct((NENT, EMB), F32),
               jax.ShapeDtypeStruct((NPT, EMB), F32)],
)


def _ent1_body(acc_c_ref, acc_d_ref, cnt0_ref, cnt1_ref, ent_ref, ec1_ref,
               ed1_ref, out_ref):
    c = jnp.clip(cnt0_ref[...][:, :1] + cnt1_ref[...][:, :1], 1.0, None)
    out_ref[...] = (2.0 * ent_ref[...] + ec1_ref[...] + ed1_ref[...]
                    + _l2(acc_c_ref[...] / c) + _l2(acc_d_ref[...] / c))


_tc_ent1 = pl.pallas_call(
    _ent1_body,
    grid=(NEB,),
    in_specs=[pl.BlockSpec((BLK, EMB), lambda i: (i, 0))] * 7,
    out_specs=pl.BlockSpec((BLK, EMB), lambda i: (i, 0)),
    out_shape=jax.ShapeDtypeStruct((NENT, EMB), F32),
)


def _user_branch(ua, u_prev, lat, w, p):
    w1, b1, uaW, uab, w2, b2, waW, wab = p
    t1u = _mmT(u_prev, w1) + b1
    t1l = _mmT(lat, w1) + b1
    score = _softmax(_leaky(_mmT(_mmT(t1u, t1l), uaW) + uab))
    t2l = _mmT(lat, w2) + b2
    t2w = _mmT(w, w2) + b2
    sa = _mmT(_mmT(t2l, t2w), waW) + wab
    lat_new = _mm(_softmax(_leaky(sa)), w)
    gate = _mm(score, lat_new)
    return _l2(ua * gate + ua), lat_new


def _user0_body(im_ref, e_ref, u_ref, latc_ref, latd_ref, w_ref, wd_ref,
                *rest):
    pc = [r[...] for r in rest[:8]]
    pd = [r[...] for r in rest[8:16]]
    uc_ref, ud_ref, lc_ref, ld_ref, cor_ref = rest[16:]
    ua = _mm(im_ref[...], e_ref[...])
    u_prev = u_ref[...]
    uc, lc = _user_branch(ua, u_prev, latc_ref[...], w_ref[...], pc)
    ud, ld = _user_branch(ua, u_prev, latd_ref[...], wd_ref[...], pd)
    uc_ref[...] = uc
    ud_ref[...] = ud
    lc_ref[...] = lc
    ld_ref[...] = ld
    w = w_ref[...]
    wT = w.T
    nrm = jnp.sqrt(jnp.sum(wT * wT, axis=1, keepdims=True))
    ndT = wT / nrm
    pos = jnp.sum(ndT * ndT, axis=1)
    ttl = jnp.sum(_mm(wT, w), axis=1)
    cor_ref[...] = jnp.full((1, 1), 0.0, F32) + jnp.sum(ttl - pos) / TEMP


def _pspecs():
    return [pl.BlockSpec()] * 8


_tc_user0 = pl.pallas_call(
    _user0_body,
    grid=(NUB,),
    in_specs=[pl.BlockSpec((UB, NENT), lambda i: (i, 0)),
              pl.BlockSpec((NENT, EMB), lambda i: (0, 0)),
              pl.BlockSpec((UB, EMB), lambda i: (i, 0)),
              pl.BlockSpec((NF, EMB), lambda i: (0, 0)),
              pl.BlockSpec((NF, EMB), lambda i: (0, 0)),
              pl.BlockSpec((R1, EMB), lambda i: (0, 0)),
              pl.BlockSpec((R2, EMB), lambda i: (0, 0))]
    + _pspecs() + _pspecs(),
    out_specs=[pl.BlockSpec((UB, EMB), lambda i: (i, 0)),
               pl.BlockSpec((UB, EMB), lambda i: (i, 0)),
               pl.BlockSpec((NF, EMB), lambda i: (0, 0)),
               pl.BlockSpec((NF, EMB), lambda i: (0, 0)),
               pl.BlockSpec((1, 1), lambda i: (0, 0))],
    out_shape=[jax.ShapeDtypeStruct((NUSR, EMB), F32),
               jax.ShapeDtypeStruct((NUSR, EMB), F32),
               jax.ShapeDtypeStruct((NF, EMB), F32),
               jax.ShapeDtypeStruct((NF, EMB), F32),
               jax.ShapeDtypeStruct((1, 1), F32)],
)


def _user1_body(im_ref, ec_ref, ed_ref, uc1_ref, ud1_ref, lc1_ref, ld1_ref,
                w_ref, wd_ref, uemb_ref, *rest):
    pc = [r[...] for r in rest[:8]]
    pd = [r[...] for r in rest[8:16]]
    out_ref = rest[16]
    im = im_ref[...]
    uac = _mm(im, ec_ref[...])
    uad = _mm(im, ed_ref[...])
    uc2, _ = _user_branch(uac, uc1_ref[...], lc1_ref[...], w_ref[...], pc)
    ud2, _ = _user_branch(uad, ud1_ref[...], ld1_ref[...], wd_ref[...], pd)
    out_ref[...] = (2.0 * uemb_ref[...] + uc1_ref[...] + ud1_ref[...]
                    + uc2 + ud2)


_tc_user1 = pl.pallas_call(
    _user1_body,
    grid=(NUB,),
    in_specs=[pl.BlockSpec((UB, NENT), lambda i: (i, 0)),
              pl.BlockSpec((NENT, EMB), lambda i: (0, 0)),
              pl.BlockSpec((NENT, EMB), lambda i: (0, 0)),
              pl.BlockSpec((UB, EMB), lambda i: (i, 0)),
              pl.BlockSpec((UB, EMB), lambda i: (i, 0)),
              pl.BlockSpec((NF, EMB), lambda i: (0, 0)),
              pl.BlockSpec((NF, EMB), lambda i: (0, 0)),
              pl.BlockSpec((R1, EMB), lambda i: (0, 0)),
              pl.BlockSpec((R2, EMB), lambda i: (0, 0)),
              pl.BlockSpec((UB, EMB), lambda i: (i, 0))]
    + _pspecs() + _pspecs(),
    out_specs=pl.BlockSpec((UB, EMB), lambda i: (i, 0)),
    out_shape=jax.ShapeDtypeStruct((NUSR, EMB), F32),
)


def _params(p, i):
    return (p["w1_W"][i], p["w1_b"][i].reshape(1, EMB),
            p["ua_W"][i], p["ua_b"][i].reshape(1, NF),
            p["w2_W"][i], p["w2_b"][i].reshape(1, EMB),
            p["wa_W"][i], p["wa_b"][i].reshape(1, -1))


def kernel(user_emb, entity_emb, latent_emb, latent_div_emb, interact_mat,
           weight, weight_d, params_c, params_d, edge_index, edge_type,
           entity_cate_set):
    head = edge_index[0]
    tail = edge_index[1]
    g0 = ((edge_type.astype(I32) - 1) * NENT + tail).reshape(NCH, CHUNK)
    g1 = (R1 * NENT + tail).reshape(NCH, CHUNK)
    h2 = head.reshape(NCH, CHUNK)
    cate2 = entity_cate_set.reshape(NENT, 1)
    ze = jnp.zeros((STRIPE, EMB), F32)
    on = jnp.ones((CHUNK, EMB), F32)

    sc = _make_sc()
    sc_cnt = _make_sc_cnt()

    w3 = weight.reshape(R1, 1, EMB)
    p0 = _tc_prep(entity_emb, w3, weight_d, cate2)
    cnt0, cnt1 = sc_cnt(h2, ze, on)
    acc_c0, acc_d0 = sc(p0, g0, g1, h2, ze)
    e_c1, e_d1, p1 = _tc_ent0(acc_c0, acc_d0, cnt0, cnt1, w3, weight_d,
                              cate2)
    u_c1, u_d1, l_c1, l_d1, cor = _tc_user0(
        interact_mat, entity_emb, user_emb, latent_emb, latent_div_emb,
        weight, weight_d, *_params(params_c, 0), *_params(params_d, 0))
    acc_c1, acc_d1 = sc(p1, g0, g1, h2, ze)
    e_fin = _tc_ent1(acc_c1, acc_d1, cnt0, cnt1, entity_emb, e_c1, e_d1)
    u_fin = _tc_user1(interact_mat, e_c1, e_d1, u_c1, u_d1, l_c1, l_d1,
                      weight, weight_d, user_emb,
                      *_params(params_c, 1), *_params(params_d, 1))
    return (e_fin, u_fin, cor[0, 0])


# speedup vs baseline: 9.4904x; 1.0687x over previous
import functools

import jax
import jax.numpy as jnp
from jax import lax
from jax.experimental import pallas as pl
from jax.experimental.pallas import tpu as pltpu
from jax.experimental.pallas import tpu_sc as plsc

F32 = jnp.float32
I32 = jnp.int32

NENT = 10000
NUSR = 1024
NEDGE = 320000
EMB = 128
NF = 4
NREL = 10
NCAT = 20
R1 = NREL - 1
R2 = NCAT + NREL - 1
TEMP = 0.2

NC = 2
NS = 16
CHUNK = 80
NCH = NEDGE // CHUNK
IBATCH = 16
NBT = NCH // IBATCH
KMAX = (NBT + NS - 1) // NS
STRIPE = 640
LSTRIPE = NENT - (NS - 1) * STRIPE
BLK = 2000
UB = 128
NEB = NENT // BLK
NUB = NUSR // UB


def _sc_body(tbl, g0, g1, h2, ze, acc_c_o, acc_d_o,
             gidx, hidx, rows0, rows1, acc_sh, sem0, sem1, ssem0, ssem1):
    core = lax.axis_index("c")
    sid = lax.axis_index("s")
    row0 = sid * STRIPE
    last = NS - 1

    @pl.when(sid < last)
    def _():
        pltpu.sync_copy(ze, acc_sh.at[pl.ds(row0, STRIPE)])

    @pl.when(sid == last)
    def _():
        pltpu.sync_copy(ze.at[pl.ds(0, LSTRIPE)],
                        acc_sh.at[pl.ds(last * STRIPE, LSTRIPE)])

    plsc.subcore_barrier()
    rows = (rows0, rows1)
    sems = (sem0, sem1)
    ssems = (ssem0, ssem1)

    def batch(k, carry):
        b = k * NS + sid

        @pl.when(b < NBT)
        def _():
            base = b * IBATCH

            @pl.when(core == 0)
            def _():
                pltpu.sync_copy(g0.at[pl.ds(base, IBATCH)], gidx)

            @pl.when(core == 1)
            def _():
                pltpu.sync_copy(g1.at[pl.ds(base, IBATCH)], gidx)

            pltpu.sync_copy(h2.at[pl.ds(base, IBATCH)], hidx)

            descs = [None, None]
            sdescs = [None, None]
            descs[0] = pltpu.async_copy(tbl.at[gidx.at[0]], rows0, sem0)
            for j in range(IBATCH):
                if j + 1 < IBATCH:
                    if sdescs[(j + 1) % 2] is not None:
                        sdescs[(j + 1) % 2].wait()
                    descs[(j + 1) % 2] = pltpu.async_copy(
                        tbl.at[gidx.at[j + 1]], rows[(j + 1) % 2],
                        sems[(j + 1) % 2])
                descs[j % 2].wait()
                sdescs[j % 2] = pltpu.async_copy(
                    rows[j % 2], acc_sh.at[hidx.at[j]], ssems[j % 2],
                    add=True)
            sdescs[0].wait()
            sdescs[1].wait()

        return carry

    lax.fori_loop(0, KMAX, batch, 0)
    plsc.subcore_barrier()

    @pl.when((core == 0) & (sid < last))
    def _():
        pltpu.sync_copy(acc_sh.at[pl.ds(row0, STRIPE)],
                        acc_c_o.at[pl.ds(row0, STRIPE)])

    @pl.when((core == 0) & (sid == last))
    def _():
        pltpu.sync_copy(acc_sh.at[pl.ds(last * STRIPE, LSTRIPE)],
                        acc_c_o.at[pl.ds(last * STRIPE, LSTRIPE)])

    @pl.when((core == 1) & (sid < last))
    def _():
        pltpu.sync_copy(acc_sh.at[pl.ds(row0, STRIPE)],
                        acc_d_o.at[pl.ds(row0, STRIPE)])

    @pl.when((core == 1) & (sid == last))
    def _():
        pltpu.sync_copy(acc_sh.at[pl.ds(last * STRIPE, LSTRIPE)],
                        acc_d_o.at[pl.ds(last * STRIPE, LSTRIPE)])


NBTC = NBT // NC
KMAXC = (NBTC + NS - 1) // NS


def _sc_cnt_body(h2, zc, on, cnt0_o, cnt1_o, hidx, ones, cnt_sh):
    core = lax.axis_index("c")
    sid = lax.axis_index("s")
    row0 = sid * STRIPE
    last = NS - 1

    @pl.when(sid < last)
    def _():
        pltpu.sync_copy(zc, cnt_sh.at[pl.ds(row0, STRIPE)])

    @pl.when(sid == last)
    def _():
        pltpu.sync_copy(zc.at[pl.ds(0, LSTRIPE)],
                        cnt_sh.at[pl.ds(last * STRIPE, LSTRIPE)])

    pltpu.sync_copy(on, ones)
    plsc.subcore_barrier()

    def batch(k, carry):
        kk = k * NS + sid

        @pl.when(kk < NBTC)
        def _():
            b = core * NBTC + kk
            pltpu.sync_copy(h2.at[pl.ds(b * IBATCH, IBATCH)], hidx)

            def chunk(j, c):
                pltpu.sync_copy(ones, cnt_sh.at[hidx.at[j]], add=True)
                return c

            lax.fori_loop(0, IBATCH, chunk, 0)

        return carry

    lax.fori_loop(0, KMAXC, batch, 0)
    plsc.subcore_barrier()

    @pl.when((core == 0) & (sid < last))
    def _():
        pltpu.sync_copy(cnt_sh.at[pl.ds(row0, STRIPE)],
                        cnt0_o.at[pl.ds(row0, STRIPE)])

    @pl.when((core == 0) & (sid == last))
    def _():
        pltpu.sync_copy(cnt_sh.at[pl.ds(last * STRIPE, LSTRIPE)],
                        cnt0_o.at[pl.ds(last * STRIPE, LSTRIPE)])

    @pl.when((core == 1) & (sid < last))
    def _():
        pltpu.sync_copy(cnt_sh.at[pl.ds(row0, STRIPE)],
                        cnt1_o.at[pl.ds(row0, STRIPE)])

    @pl.when((core == 1) & (sid == last))
    def _():
        pltpu.sync_copy(cnt_sh.at[pl.ds(last * STRIPE, LSTRIPE)],
                        cnt1_o.at[pl.ds(last * STRIPE, LSTRIPE)])


def _make_sc():
    mesh = plsc.VectorSubcoreMesh(core_axis_name="c", subcore_axis_name="s",
                                  num_cores=NC, num_subcores=NS)
    outs = (jax.ShapeDtypeStruct((NENT, EMB), F32),
            jax.ShapeDtypeStruct((NENT, EMB), F32))
    scratch = [pltpu.VMEM((IBATCH, CHUNK), I32),
               pltpu.VMEM((IBATCH, CHUNK), I32),
               pltpu.VMEM((CHUNK, EMB), F32),
               pltpu.VMEM((CHUNK, EMB), F32),
               pltpu.VMEM_SHARED((NENT, EMB), F32),
               pltpu.SemaphoreType.DMA,
               pltpu.SemaphoreType.DMA,
               pltpu.SemaphoreType.DMA,
               pltpu.SemaphoreType.DMA]
    return pl.kernel(_sc_body, out_type=outs, mesh=mesh,
                     scratch_types=scratch)


def _make_sc_cnt():
    mesh = plsc.VectorSubcoreMesh(core_axis_name="c", subcore_axis_name="s",
                                  num_cores=NC, num_subcores=NS)
    outs = (jax.ShapeDtypeStruct((NENT, EMB), F32),
            jax.ShapeDtypeStruct((NENT, EMB), F32))
    scratch = [pltpu.VMEM((IBATCH, CHUNK), I32),
               pltpu.VMEM((CHUNK, EMB), F32),
               pltpu.VMEM_SHARED((NENT, EMB), F32)]
    return pl.kernel(_sc_cnt_body, out_type=outs, mesh=mesh,
                     scratch_types=scratch)


def _mm(a, b):
    return lax.dot_general(a, b, (((1,), (0,)), ((), ())),
                           preferred_element_type=F32)


def _mmT(a, b):
    return lax.dot_general(a, b, (((1,), (1,)), ((), ())),
                           preferred_element_type=F32)


def _l2(x):
    return x / jnp.clip(jnp.sqrt(jnp.sum(x * x, axis=1, keepdims=True)),
                        1e-12, None)


def _leaky(x):
    return jnp.where(x >= 0, x, 0.2 * x)


def _softmax(x):
    m = jnp.max(x, axis=-1, keepdims=True)
    e = jnp.exp(x - m)
    return e / jnp.sum(e, axis=-1, keepdims=True)


def _onehot_wd(cate, wd):
    oh = (cate - 1 == lax.broadcasted_iota(I32, (cate.shape[0], R2), 1))
    return _mm(oh.astype(F32), wd)


NPT = (R1 + 1) * NENT


def _prep_body(e_ref, w_ref, wd_ref, cate_ref, p_ref):
    r = pl.program_id(0)
    e = e_ref[...]

    @pl.when(r < R1)
    def _():
        p_ref[...] = e * w_ref[0]

    @pl.when(r == R1)
    def _():
        p_ref[...] = e * _onehot_wd(cate_ref[...], wd_ref[...])


_tc_prep = pl.pallas_call(
    _prep_body,
    grid=(R1 + 1, NEB),
    in_specs=[pl.BlockSpec((BLK, EMB), lambda r, i: (i, 0)),
              pl.BlockSpec((1, 1, EMB), lambda r, i: (jnp.minimum(r, R1 - 1),
                                                      0, 0)),
              pl.BlockSpec((R2, EMB), lambda r, i: (0, 0)),
              pl.BlockSpec((BLK, 1), lambda r, i: (i, 0))],
    out_specs=pl.BlockSpec((BLK, EMB), lambda r, i: (r * NEB + i, 0)),
    out_shape=jax.ShapeDtypeStruct((NPT, EMB), F32),
)


def _ent0_body(acc_c_ref, acc_d_ref, cnt0_ref, cnt1_ref, w_ref, wd_ref,
               cate_ref, ec_ref, ed_ref, p_ref):
    r = pl.program_id(0)
    c = jnp.clip(cnt0_ref[...][:, :1] + cnt1_ref[...][:, :1], 1.0, None)
    ec = _l2(acc_c_ref[...] / c)
    ed = _l2(acc_d_ref[...] / c)
    ec_ref[...] = ec
    ed_ref[...] = ed

    @pl.when(r < R1)
    def _():
        p_ref[...] = ec * w_ref[0]

    @pl.when(r == R1)
    def _():
        p_ref[...] = ed * _onehot_wd(cate_ref[...], wd_ref[...])


_tc_ent0 = pl.pallas_call(
    _ent0_body,
    grid=(R1 + 1, NEB),
    in_specs=[pl.BlockSpec((BLK, EMB), lambda r, i: (i, 0)),
              pl.BlockSpec((BLK, EMB), lambda r, i: (i, 0)),
              pl.BlockSpec((BLK, EMB), lambda r, i: (i, 0)),
              pl.BlockSpec((BLK, EMB), lambda r, i: (i, 0)),
              pl.BlockSpec((1, 1, EMB), lambda r, i: (jnp.minimum(r, R1 - 1),
                                                      0, 0)),
              pl.BlockSpec((R2, EMB), lambda r, i: (0, 0)),
              pl.BlockSpec((BLK, 1), lambda r, i: (i, 0))],
    out_specs=[pl.BlockSpec((BLK, EMB), lambda r, i: (i, 0)),
               pl.BlockSpec((BLK, EMB), lambda r, i: (i, 0)),
               pl.BlockSpec((BLK, EMB), lambda r, i: (r * NEB + i, 0))],
    out_shape=[jax.ShapeDtypeStruct((NENT, EMB), F32),
               jax.ShapeDtypeStruct((NENT, EMB), F32),
               jax.ShapeDtypeStruct((NPT, EMB), F32)],
)


def _ent1_body(acc_c_ref, acc_d_ref, cnt0_ref, cnt1_ref, ent_ref, ec1_ref,
               ed1_ref, out_ref):
    c = jnp.clip(cnt0_ref[...][:, :1] + cnt1_ref[...][:, :1], 1.0, None)
    out_ref[...] = (2.0 * ent_ref[...] + ec1_ref[...] + ed1_ref[...]
                    + _l2(acc_c_ref[...] / c) + _l2(acc_d_ref[...] / c))


_tc_ent1 = pl.pallas_call(
    _ent1_body,
    grid=(NEB,),
    in_specs=[pl.BlockSpec((BLK, EMB), lambda i: (i, 0))] * 7,
    out_specs=pl.BlockSpec((BLK, EMB), lambda i: (i, 0)),
    out_shape=jax.ShapeDtypeStruct((NENT, EMB), F32),
)


def _user_branch(ua, u_prev, lat, w, p):
    w1, b1, uaW, uab, w2, b2, waW, wab = p
    t1u = _mmT(u_prev, w1) + b1
    t1l = _mmT(lat, w1) + b1
    score = _softmax(_leaky(_mmT(_mmT(t1u, t1l), uaW) + uab))
    t2l = _mmT(lat, w2) + b2
    t2w = _mmT(w, w2) + b2
    sa = _mmT(_mmT(t2l, t2w), waW) + wab
    lat_new = _mm(_softmax(_leaky(sa)), w)
    gate = _mm(score, lat_new)
    return _l2(ua * gate + ua), lat_new


def _user0_body(im_ref, e_ref, u_ref, latc_ref, latd_ref, w_ref, wd_ref,
                *rest):
    pc = [r[...] for r in rest[:8]]
    pd = [r[...] for r in rest[8:16]]
    uc_ref, ud_ref, lc_ref, ld_ref, cor_ref = rest[16:]
    ua = _mm(im_ref[...], e_ref[...])
    u_prev = u_ref[...]
    uc, lc = _user_branch(ua, u_prev, latc_ref[...], w_ref[...], pc)
    ud, ld = _user_branch(ua, u_prev, latd_ref[...], wd_ref[...], pd)
    uc_ref[...] = uc
    ud_ref[...] = ud
    lc_ref[...] = lc
    ld_ref[...] = ld
    w = w_ref[...]
    wT = w.T
    nrm = jnp.sqrt(jnp.sum(wT * wT, axis=1, keepdims=True))
    ndT = wT / nrm
    pos = jnp.sum(ndT * ndT, axis=1)
    ttl = jnp.sum(_mm(wT, w), axis=1)
    cor_ref[...] = jnp.full((1, 1), 0.0, F32) + jnp.sum(ttl - pos) / TEMP


def _pspecs():
    return [pl.BlockSpec()] * 8


_tc_user0 = pl.pallas_call(
    _user0_body,
    grid=(NUB,),
    in_specs=[pl.BlockSpec((UB, NENT), lambda i: (i, 0)),
              pl.BlockSpec((NENT, EMB), lambda i: (0, 0)),
              pl.BlockSpec((UB, EMB), lambda i: (i, 0)),
              pl.BlockSpec((NF, EMB), lambda i: (0, 0)),
              pl.BlockSpec((NF, EMB), lambda i: (0, 0)),
              pl.BlockSpec((R1, EMB), lambda i: (0, 0)),
              pl.BlockSpec((R2, EMB), lambda i: (0, 0))]
    + _pspecs() + _pspecs(),
    out_specs=[pl.BlockSpec((UB, EMB), lambda i: (i, 0)),
               pl.BlockSpec((UB, EMB), lambda i: (i, 0)),
               pl.BlockSpec((NF, EMB), lambda i: (0, 0)),
               pl.BlockSpec((NF, EMB), lambda i: (0, 0)),
               pl.BlockSpec((1, 1), lambda i: (0, 0))],
    out_shape=[jax.ShapeDtypeStruct((NUSR, EMB), F32),
               jax.ShapeDtypeStruct((NUSR, EMB), F32),
               jax.ShapeDtypeStruct((NF, EMB), F32),
               jax.ShapeDtypeStruct((NF, EMB), F32),
               jax.ShapeDtypeStruct((1, 1), F32)],
)


def _user1_body(im_ref, ec_ref, ed_ref, uc1_ref, ud1_ref, lc1_ref, ld1_ref,
                w_ref, wd_ref, uemb_ref, *rest):
    pc = [r[...] for r in rest[:8]]
    pd = [r[...] for r in rest[8:16]]
    out_ref = rest[16]
    im = im_ref[...]
    uac = _mm(im, ec_ref[...])
    uad = _mm(im, ed_ref[...])
    uc2, _ = _user_branch(uac, uc1_ref[...], lc1_ref[...], w_ref[...], pc)
    ud2, _ = _user_branch(uad, ud1_ref[...], ld1_ref[...], wd_ref[...], pd)
    out_ref[...] = (2.0 * uemb_ref[...] + uc1_ref[...] + ud1_ref[...]
                    + uc2 + ud2)


_tc_user1 = pl.pallas_call(
    _user1_body,
    grid=(NUB,),
    in_specs=[pl.BlockSpec((UB, NENT), lambda i: (i, 0)),
              pl.BlockSpec((NENT, EMB), lambda i: (0, 0)),
              pl.BlockSpec((NENT, EMB), lambda i: (0, 0)),
              pl.BlockSpec((UB, EMB), lambda i: (i, 0)),
              pl.BlockSpec((UB, EMB), lambda i: (i, 0)),
              pl.BlockSpec((NF, EMB), lambda i: (0, 0)),
              pl.BlockSpec((NF, EMB), lambda i: (0, 0)),
              pl.BlockSpec((R1, EMB), lambda i: (0, 0)),
              pl.BlockSpec((R2, EMB), lambda i: (0, 0)),
              pl.BlockSpec((UB, EMB), lambda i: (i, 0))]
    + _pspecs() + _pspecs(),
    out_specs=pl.BlockSpec((UB, EMB), lambda i: (i, 0)),
    out_shape=jax.ShapeDtypeStruct((NUSR, EMB), F32),
)


def _params(p, i):
    return (p["w1_W"][i], p["w1_b"][i].reshape(1, EMB),
            p["ua_W"][i], p["ua_b"][i].reshape(1, NF),
            p["w2_W"][i], p["w2_b"][i].reshape(1, EMB),
            p["wa_W"][i], p["wa_b"][i].reshape(1, -1))


def kernel(user_emb, entity_emb, latent_emb, latent_div_emb, interact_mat,
           weight, weight_d, params_c, params_d, edge_index, edge_type,
           entity_cate_set):
    head = edge_index[0]
    tail = edge_index[1]
    g0 = ((edge_type.astype(I32) - 1) * NENT + tail).reshape(NCH, CHUNK)
    g1 = (R1 * NENT + tail).reshape(NCH, CHUNK)
    h2 = head.reshape(NCH, CHUNK)
    cate2 = entity_cate_set.reshape(NENT, 1)
    ze = jnp.zeros((STRIPE, EMB), F32)
    on = jnp.ones((CHUNK, EMB), F32)

    sc = _make_sc()
    sc_cnt = _make_sc_cnt()

    w3 = weight.reshape(R1, 1, EMB)
    p0 = _tc_prep(entity_emb, w3, weight_d, cate2)
    cnt0, cnt1 = sc_cnt(h2, ze, on)
    acc_c0, acc_d0 = sc(p0, g0, g1, h2, ze)
    e_c1, e_d1, p1 = _tc_ent0(acc_c0, acc_d0, cnt0, cnt1, w3, weight_d,
                              cate2)
    u_c1, u_d1, l_c1, l_d1, cor = _tc_user0(
        interact_mat, entity_emb, user_emb, latent_emb, latent_div_emb,
        weight, weight_d, *_params(params_c, 0), *_params(params_d, 0))
    acc_c1, acc_d1 = sc(p1, g0, g1, h2, ze)
    e_fin = _tc_ent1(acc_c1, acc_d1, cnt0, cnt1, entity_emb, e_c1, e_d1)
    u_fin = _tc_user1(interact_mat, e_c1, e_d1, u_c1, u_d1, l_c1, l_d1,
                      weight, weight_d, user_emb,
                      *_params(params_c, 1), *_params(params_d, 1))
    return (e_fin, u_fin, cor[0, 0])
